# baseline (speedup 1.0000x reference)
import functools

import numpy as np

import jax
from jax import lax
import jax.numpy as jnp
from jax.experimental import pallas as pl
from jax.experimental.pallas import tpu as pltpu
from jax.experimental.pallas import tpu_sc as plsc

_N = 10000
_E = 320000
_DIN = 128
_DH = 64
_DOUT = 64

_W1 = 96
_W2 = 64
_DT = jnp.bfloat16
_NTILES = 32
_EPT = _E // _NTILES
_CB = 100
_NCH = _EPT // _CB
_NBUF = 4
_NPAD = 10240
_RPT = _NPAD // 16
_ZR = 320


_SIG = np.array([(m // 32) * 32 + ((m % 32) // 2 if m % 2 == 0 else
                                   16 + (m % 32 - 1) // 2)
                 for m in range(_DH)], dtype=np.int32)
_PMAT = np.zeros((_DH, _DH), np.float32)
_PMAT[_SIG, np.arange(_DH)] = 1.0


@functools.cache
def _make_sc_segsum(W):
  mesh = plsc.VectorSubcoreMesh(core_axis_name="c", subcore_axis_name="s")

  WO = 80 if W == 96 else W

  @functools.partial(
      pl.kernel,
      out_type=jax.ShapeDtypeStruct((2, _NPAD, WO), jnp.float32),
      mesh=mesh,
      compiler_params=pltpu.CompilerParams(
          use_tc_tiling_on_sc=False, needs_layout_passes=False),
      scratch_types=[
          pltpu.VMEM((_NCH, _CB), jnp.int32),
          pltpu.VMEM((_NCH, _CB), jnp.int32),
          pltpu.VMEM((_CB, W), _DT),
          pltpu.VMEM((_CB, W), _DT),
          pltpu.VMEM((_CB, W), _DT),
          pltpu.VMEM((_CB, W), _DT),
          pltpu.VMEM((_ZR, W), _DT),
          pltpu.VMEM((_ZR, WO), jnp.float32),
          pltpu.VMEM_SHARED((_NPAD, W), _DT),
          pltpu.SemaphoreType.DMA,
          pltpu.SemaphoreType.DMA,
          pltpu.SemaphoreType.DMA,
          pltpu.SemaphoreType.DMA,
      ],
  )
  def sc(table, ei, out, srcv, dstv, r0, r1, r2, r3, zb, fb, acc,
         s0, s1, s2, s3):
    rows = (r0, r1, r2, r3)
    sems = (s0, s1, s2, s3)
    cid = lax.axis_index("c")
    sid = lax.axis_index("s")
    wid = sid * 2 + cid

    pltpu.sync_copy(ei.at[0, wid], srcv)
    pltpu.sync_copy(ei.at[1, wid], dstv)

    zvec = jnp.zeros((32,), _DT)

    def zrow(i, carry):
      for j in range(W // 32):
        zb[i, pl.ds(j * 32, 32)] = zvec
      return carry

    lax.fori_loop(0, _ZR, zrow, 0)
    row0 = sid * _RPT
    for k in range(_RPT // _ZR):
      pltpu.sync_copy(zb, acc.at[pl.ds(row0 + k * _ZR, _ZR)])
    plsc.subcore_barrier()

    def gather(c, b):
      pltpu.async_copy(table.at[srcv.at[c]], rows[b], sems[b])

    def scatter(c, b):
      pltpu.make_async_copy(table.at[srcv.at[c]], rows[b], sems[b]).wait()
      pltpu.sync_copy(rows[b], acc.at[dstv.at[c]], add=True)

    for b in range(_NBUF):
      gather(b, b)

    def block(j, carry):
      for b in range(_NBUF):
        c = j * _NBUF + b
        scatter(c, b)
        gather(c + _NBUF, b)
      return carry

    lax.fori_loop(0, _NCH // _NBUF - 1, block, 0)
    for b in range(_NBUF):
      scatter(_NCH - _NBUF + b, b)
    plsc.subcore_barrier()

    hi_mask = jnp.full((16,), 0xFFFF0000, jnp.uint32)
    shift16 = jnp.full((16,), 16, jnp.uint32)

    def crow(i4, carry):
      for u in range(4):
        i = i4 * 4 + u
        for j in range(W // 32):
          v = zb[i, pl.ds(j * 32, 32)]
          xi = plsc.bitcast(v, jnp.uint32)
          fb[i, pl.ds(j * 32, 16)] = plsc.bitcast(
              jnp.left_shift(xi, shift16), jnp.float32)
          if j * 32 + 16 < WO:
            fb[i, pl.ds(j * 32 + 16, 16)] = plsc.bitcast(
                jnp.bitwise_and(xi, hi_mask), jnp.float32)
      return carry

    for k in range(_RPT // _ZR):
      r = row0 + k * _ZR
      pltpu.sync_copy(acc.at[pl.ds(r, _ZR)], zb)
      lax.fori_loop(0, _ZR // 4, crow, 0)
      pltpu.sync_copy(fb, out.at[cid, pl.ds(r, _ZR)])

  return sc


def _tc_pre(x, Wl1, Wr1, pm):
  def body(x_ref, wl_ref, wr_ref, pm_ref, t_ref, r_ref):
    xv = x_ref[...]
    wlp = jnp.dot(wl_ref[...], pm_ref[...],
                  preferred_element_type=jnp.float32)
    p = jnp.dot(xv, wlp, preferred_element_type=jnp.float32)
    t_ref[...] = jnp.concatenate(
        [p.astype(_DT), jnp.ones((_N, _W1 - _DH), _DT)], axis=1)
    r_ref[...] = jnp.dot(xv, wr_ref[...], preferred_element_type=jnp.float32)

  return pl.pallas_call(
      body,
      out_shape=(jax.ShapeDtypeStruct((_N, _W1), _DT),
                 jax.ShapeDtypeStruct((_N, _DH), jnp.float32)),
  )(x, Wl1, Wr1, pm)


def _tc_mid(aggp, r1, b1, Wl2, Wr2, pm):
  def body(a_ref, r1_ref, b1_ref, wl_ref, wr_ref, pm_ref, t2_ref, r2_ref,
           inv_ref):
    acc = a_ref[0, :_N, :] + a_ref[1, :_N, :]
    cnt = acc[:, _DH:_DH + 1]
    inv = 1.0 / jnp.maximum(cnt, 1.0)
    o = acc[:, :_DH] * inv + b1_ref[...] + r1_ref[...]
    nrm = jnp.sqrt(jnp.sum(o * o, axis=1, keepdims=True))
    h = jnp.maximum(o / jnp.maximum(nrm, 1e-12), 0.0)
    wlp = jnp.dot(wl_ref[...], pm_ref[...],
                  preferred_element_type=jnp.float32)
    t2_ref[...] = jnp.dot(
        h, wlp, preferred_element_type=jnp.float32).astype(_DT)
    r2_ref[...] = jnp.dot(h, wr_ref[...], preferred_element_type=jnp.float32)
    inv_ref[...] = inv

  return pl.pallas_call(
      body,
      out_shape=(jax.ShapeDtypeStruct((_N, _W2), _DT),
                 jax.ShapeDtypeStruct((_N, _DOUT), jnp.float32),
                 jax.ShapeDtypeStruct((_N, 1), jnp.float32)),
  )(aggp, r1, b1.reshape(1, _DH), Wl2, Wr2, pm)


def _tc_post(aggp2, r2, inv, b2):
  def body(a_ref, r2_ref, inv_ref, b2_ref, out_ref):
    acc = a_ref[0, :_N, :] + a_ref[1, :_N, :]
    o = acc * inv_ref[...] + b2_ref[...] + r2_ref[...]
    nrm = jnp.sqrt(jnp.sum(o * o, axis=1, keepdims=True))
    o = o / jnp.maximum(nrm, 1e-12)
    out_ref[...] = jnp.where(o > 0.0, o, jnp.exp(jnp.minimum(o, 0.0)) - 1.0)

  return pl.pallas_call(
      body,
      out_shape=jax.ShapeDtypeStruct((_N, _DOUT), jnp.float32),
  )(aggp2, r2, inv, b2.reshape(1, _DOUT))


def kernel(x, edge_index, Wl1, b1, Wr1, Wl2, b2, Wr2):
  ei = edge_index.reshape(2, _NTILES, _NCH, _CB)
  pm = jnp.asarray(_PMAT)
  table1, r1 = _tc_pre(x, Wl1, Wr1, pm)
  aggp1 = _make_sc_segsum(_W1)(table1, ei)
  table2, r2, inv = _tc_mid(aggp1, r1, b1, Wl2, Wr2, pm)
  aggp2 = _make_sc_segsum(_W2)(table2, ei)
  return _tc_post(aggp2, r2, inv, b2)


# speedup vs baseline: 1.0451x; 1.0451x over previous
import functools

import jax
from jax import lax
import jax.numpy as jnp
from jax.experimental import pallas as pl
from jax.experimental.pallas import tpu as pltpu
from jax.experimental.pallas import tpu_sc as plsc

_N = 10000
_E = 320000
_DIN = 128
_DH = 64
_DOUT = 64

_W1 = 96
_W2 = 64
_DT = jnp.bfloat16
_NTILES = 32
_EPT = _E // _NTILES
_CB = 125
_NCH = _EPT // _CB
_NBUF = 8
_NPAD = 10240
_RPT = _NPAD // 16
_ZR = 320


@functools.cache
def _make_sc_segsum(W):
  mesh = plsc.VectorSubcoreMesh(core_axis_name="c", subcore_axis_name="s")

  @functools.partial(
      pl.kernel,
      out_type=jax.ShapeDtypeStruct((2, _NPAD, W), _DT),
      mesh=mesh,
      compiler_params=pltpu.CompilerParams(use_tc_tiling_on_sc=False),
      scratch_types=[
          pltpu.VMEM((_NCH, _CB), jnp.int32),
          pltpu.VMEM((_NCH, _CB), jnp.int32),
          pltpu.VMEM((_CB, W), _DT),
          pltpu.VMEM((_CB, W), _DT),
          pltpu.VMEM((_CB, W), _DT),
          pltpu.VMEM((_CB, W), _DT),
          pltpu.VMEM((_CB, W), _DT),
          pltpu.VMEM((_CB, W), _DT),
          pltpu.VMEM((_CB, W), _DT),
          pltpu.VMEM((_CB, W), _DT),
          pltpu.VMEM((_ZR, W), _DT),
          pltpu.VMEM_SHARED((_NPAD, W), _DT),
          pltpu.SemaphoreType.DMA,
          pltpu.SemaphoreType.DMA,
          pltpu.SemaphoreType.DMA,
          pltpu.SemaphoreType.DMA,
          pltpu.SemaphoreType.DMA,
          pltpu.SemaphoreType.DMA,
          pltpu.SemaphoreType.DMA,
          pltpu.SemaphoreType.DMA,
          pltpu.SemaphoreType.DMA,
          pltpu.SemaphoreType.DMA,
          pltpu.SemaphoreType.DMA,
          pltpu.SemaphoreType.DMA,
          pltpu.SemaphoreType.DMA,
          pltpu.SemaphoreType.DMA,
          pltpu.SemaphoreType.DMA,
          pltpu.SemaphoreType.DMA,
      ],
  )
  def sc(table, ei, out, srcv, dstv, r0, r1, r2, r3, r4, r5, r6, r7, zb, acc,
         g0, g1, g2, g3, g4, g5, g6, g7, t0, t1, t2, t3, t4, t5, t6, t7):
    rows = (r0, r1, r2, r3, r4, r5, r6, r7)
    gsems = (g0, g1, g2, g3, g4, g5, g6, g7)
    ssems = (t0, t1, t2, t3, t4, t5, t6, t7)
    cid = lax.axis_index("c")
    sid = lax.axis_index("s")
    wid = sid * 2 + cid

    pltpu.sync_copy(ei.at[0, wid], srcv)
    pltpu.sync_copy(ei.at[1, wid], dstv)

    zvec = jnp.zeros((32,), _DT)

    def zrow(i, carry):
      for j in range(W // 32):
        zb[i, pl.ds(j * 32, 32)] = zvec
      return carry

    lax.fori_loop(0, _ZR, zrow, 0)
    row0 = sid * _RPT
    for k in range(_RPT // _ZR):
      pltpu.sync_copy(zb, acc.at[pl.ds(row0 + k * _ZR, _ZR)])
    plsc.subcore_barrier()

    def gather(c, b):
      pltpu.async_copy(table.at[srcv.at[c]], rows[b], gsems[b])

    def scatter(c, b):
      pltpu.make_async_copy(table.at[srcv.at[c]], rows[b], gsems[b]).wait()
      pltpu.async_copy(rows[b], acc.at[dstv.at[c]], ssems[b], add=True)

    def drain(c, b):
      pltpu.make_async_copy(rows[b], acc.at[dstv.at[c]], ssems[b]).wait()

    for b in range(_NBUF):
      gather(b, b)

    def block(j, carry):
      for b in range(_NBUF):
        scatter(j * _NBUF + b, b)
      for b in range(_NBUF):
        drain(j * _NBUF + b, b)
        gather((j + 1) * _NBUF + b, b)
      return carry

    nblk = _NCH // _NBUF
    lax.fori_loop(0, nblk - 1, block, 0)
    for b in range(_NBUF):
      scatter((nblk - 1) * _NBUF + b, b)
    for b in range(_NBUF):
      drain((nblk - 1) * _NBUF + b, b)
    plsc.subcore_barrier()

    for k in range(_RPT // _ZR):
      r = row0 + k * _ZR
      pltpu.sync_copy(acc.at[pl.ds(r, _ZR)], zb)
      pltpu.sync_copy(zb, out.at[cid, pl.ds(r, _ZR)])

  return sc


def _tc_pre(x, Wl1, Wr1):
  def body(x_ref, wl_ref, wr_ref, t_ref, r_ref):
    xv = x_ref[...]
    p = jnp.dot(xv, wl_ref[...], preferred_element_type=jnp.float32)
    t_ref[...] = jnp.concatenate(
        [p.astype(_DT), jnp.ones((_N, _W1 - _DH), _DT)], axis=1)
    r_ref[...] = jnp.dot(xv, wr_ref[...], preferred_element_type=jnp.float32)

  return pl.pallas_call(
      body,
      out_shape=(jax.ShapeDtypeStruct((_N, _W1), _DT),
                 jax.ShapeDtypeStruct((_N, _DH), jnp.float32)),
  )(x, Wl1, Wr1)


def _tc_mid(aggp, r1, b1, Wl2, Wr2):
  def body(a_ref, r1_ref, b1_ref, wl_ref, wr_ref, t2_ref, r2_ref, inv_ref):
    acc = (a_ref[0, :_N, :] + a_ref[1, :_N, :]).astype(jnp.float32)
    cnt = acc[:, _DH:_DH + 1]
    inv = 1.0 / jnp.maximum(cnt, 1.0)
    o = acc[:, :_DH] * inv + b1_ref[...] + r1_ref[...]
    nrm = jnp.sqrt(jnp.sum(o * o, axis=1, keepdims=True))
    h = jnp.maximum(o / jnp.maximum(nrm, 1e-12), 0.0)
    t2_ref[...] = jnp.dot(
        h, wl_ref[...], preferred_element_type=jnp.float32).astype(_DT)
    r2_ref[...] = jnp.dot(h, wr_ref[...], preferred_element_type=jnp.float32)
    inv_ref[...] = inv

  return pl.pallas_call(
      body,
      out_shape=(jax.ShapeDtypeStruct((_N, _W2), _DT),
                 jax.ShapeDtypeStruct((_N, _DOUT), jnp.float32),
                 jax.ShapeDtypeStruct((_N, 1), jnp.float32)),
  )(aggp, r1, b1.reshape(1, _DH), Wl2, Wr2)


def _tc_post(aggp2, r2, inv, b2):
  def body(a_ref, r2_ref, inv_ref, b2_ref, out_ref):
    acc = (a_ref[0, :_N, :] + a_ref[1, :_N, :]).astype(jnp.float32)
    o = acc * inv_ref[...] + b2_ref[...] + r2_ref[...]
    nrm = jnp.sqrt(jnp.sum(o * o, axis=1, keepdims=True))
    o = o / jnp.maximum(nrm, 1e-12)
    out_ref[...] = jnp.where(o > 0.0, o, jnp.exp(jnp.minimum(o, 0.0)) - 1.0)

  return pl.pallas_call(
      body,
      out_shape=jax.ShapeDtypeStruct((_N, _DOUT), jnp.float32),
  )(aggp2, r2, inv, b2.reshape(1, _DOUT))


def kernel(x, edge_index, Wl1, b1, Wr1, Wl2, b2, Wr2):
  ei = edge_index.reshape(2, _NTILES, _NCH, _CB)
  table1, r1 = _tc_pre(x, Wl1, Wr1)
  aggp1 = _make_sc_segsum(_W1)(table1, ei)
  table2, r2, inv = _tc_mid(aggp1, r1, b1, Wl2, Wr2)
  aggp2 = _make_sc_segsum(_W2)(table2, ei)
  return _tc_post(aggp2, r2, inv, b2)


# speedup vs baseline: 1.0581x; 1.0124x over previous
import functools

import jax
from jax import lax
import jax.numpy as jnp
from jax.experimental import pallas as pl
from jax.experimental.pallas import tpu as pltpu
from jax.experimental.pallas import tpu_sc as plsc

_N = 10000
_E = 320000
_DIN = 128
_DH = 64
_DOUT = 64

_W1 = 96
_W2 = 64
_DT = jnp.bfloat16
_NTILES = 32
_EPT = _E // _NTILES
_CB = 125
_NCH = _EPT // _CB
_NBUF = 8
_NPAD = 10000
_RPT = _NPAD // 16
_ZR = 125
_BR = 2000


@functools.cache
def _make_sc_segsum(W):
  mesh = plsc.VectorSubcoreMesh(core_axis_name="c", subcore_axis_name="s")

  @functools.partial(
      pl.kernel,
      out_type=jax.ShapeDtypeStruct((2, _NPAD, W), _DT),
      mesh=mesh,
      compiler_params=pltpu.CompilerParams(use_tc_tiling_on_sc=False),
      scratch_types=[
          pltpu.VMEM((_NCH, _CB), jnp.int32),
          pltpu.VMEM((_NCH, _CB), jnp.int32),
          pltpu.VMEM((_CB, W), _DT),
          pltpu.VMEM((_CB, W), _DT),
          pltpu.VMEM((_CB, W), _DT),
          pltpu.VMEM((_CB, W), _DT),
          pltpu.VMEM((_CB, W), _DT),
          pltpu.VMEM((_CB, W), _DT),
          pltpu.VMEM((_CB, W), _DT),
          pltpu.VMEM((_CB, W), _DT),
          pltpu.VMEM((_ZR, W), _DT),
          pltpu.VMEM_SHARED((_NPAD, W), _DT),
          pltpu.SemaphoreType.DMA,
          pltpu.SemaphoreType.DMA,
          pltpu.SemaphoreType.DMA,
          pltpu.SemaphoreType.DMA,
          pltpu.SemaphoreType.DMA,
          pltpu.SemaphoreType.DMA,
          pltpu.SemaphoreType.DMA,
          pltpu.SemaphoreType.DMA,
          pltpu.SemaphoreType.DMA,
          pltpu.SemaphoreType.DMA,
          pltpu.SemaphoreType.DMA,
          pltpu.SemaphoreType.DMA,
          pltpu.SemaphoreType.DMA,
          pltpu.SemaphoreType.DMA,
          pltpu.SemaphoreType.DMA,
          pltpu.SemaphoreType.DMA,
      ],
  )
  def sc(table, ei, out, srcv, dstv, r0, r1, r2, r3, r4, r5, r6, r7, zb, acc,
         g0, g1, g2, g3, g4, g5, g6, g7, t0, t1, t2, t3, t4, t5, t6, t7):
    rows = (r0, r1, r2, r3, r4, r5, r6, r7)
    gsems = (g0, g1, g2, g3, g4, g5, g6, g7)
    ssems = (t0, t1, t2, t3, t4, t5, t6, t7)
    cid = lax.axis_index("c")
    sid = lax.axis_index("s")
    wid = sid * 2 + cid

    pltpu.sync_copy(ei.at[0, wid], srcv)
    pltpu.sync_copy(ei.at[1, wid], dstv)

    zvec = jnp.zeros((32,), _DT)

    def zrow(i, carry):
      for j in range(W // 32):
        zb[i, pl.ds(j * 32, 32)] = zvec
      return carry

    lax.fori_loop(0, _ZR, zrow, 0)
    row0 = sid * _RPT
    for k in range(_RPT // _ZR):
      pltpu.sync_copy(zb, acc.at[pl.ds(row0 + k * _ZR, _ZR)])
    plsc.subcore_barrier()

    def gather(c, b):
      pltpu.async_copy(table.at[srcv.at[c]], rows[b], gsems[b])

    def scatter(c, b):
      pltpu.make_async_copy(table.at[srcv.at[c]], rows[b], gsems[b]).wait()
      pltpu.async_copy(rows[b], acc.at[dstv.at[c]], ssems[b], add=True)

    def drain(c, b):
      pltpu.make_async_copy(rows[b], acc.at[dstv.at[c]], ssems[b]).wait()

    for b in range(_NBUF):
      gather(b, b)

    def block(j, carry):
      for b in range(_NBUF):
        scatter(j * _NBUF + b, b)
      for b in range(_NBUF):
        drain(j * _NBUF + b, b)
        gather((j + 1) * _NBUF + b, b)
      return carry

    nblk = _NCH // _NBUF
    lax.fori_loop(0, nblk - 1, block, 0)
    for b in range(_NBUF):
      scatter((nblk - 1) * _NBUF + b, b)
    for b in range(_NBUF):
      drain((nblk - 1) * _NBUF + b, b)
    plsc.subcore_barrier()

    for k in range(_RPT // _ZR):
      r = row0 + k * _ZR
      pltpu.sync_copy(acc.at[pl.ds(r, _ZR)], zb)
      pltpu.sync_copy(zb, out.at[cid, pl.ds(r, _ZR)])

  return sc


def _tc_pre(x, Wl1, Wr1):
  def body(x_ref, wl_ref, wr_ref, t_ref, r_ref):
    xv = x_ref[...]
    p = jnp.dot(xv, wl_ref[...], preferred_element_type=jnp.float32)
    t_ref[...] = jnp.concatenate(
        [p.astype(_DT), jnp.ones((_BR, _W1 - _DH), _DT)], axis=1)
    r_ref[...] = jnp.dot(xv, wr_ref[...], preferred_element_type=jnp.float32)

  return pl.pallas_call(
      body,
      grid=(_N // _BR,),
      in_specs=[pl.BlockSpec((_BR, _DIN), lambda i: (i, 0)),
                pl.BlockSpec((_DIN, _DH), lambda i: (0, 0)),
                pl.BlockSpec((_DIN, _DH), lambda i: (0, 0))],
      out_specs=(pl.BlockSpec((_BR, _W1), lambda i: (i, 0)),
                 pl.BlockSpec((_BR, _DH), lambda i: (i, 0))),
      out_shape=(jax.ShapeDtypeStruct((_N, _W1), _DT),
                 jax.ShapeDtypeStruct((_N, _DH), jnp.float32)),
  )(x, Wl1, Wr1)


def _tc_mid(aggp, r1, b1, Wl2, Wr2):
  def body(a_ref, r1_ref, b1_ref, wl_ref, wr_ref, t2_ref, r2_ref, inv_ref):
    acc = (a_ref[0] + a_ref[1]).astype(jnp.float32)
    cnt = acc[:, _DH:_DH + 1]
    inv = 1.0 / jnp.maximum(cnt, 1.0)
    o = acc[:, :_DH] * inv + b1_ref[...] + r1_ref[...]
    nrm = jnp.sqrt(jnp.sum(o * o, axis=1, keepdims=True))
    h = jnp.maximum(o / jnp.maximum(nrm, 1e-12), 0.0)
    t2_ref[...] = jnp.dot(
        h, wl_ref[...], preferred_element_type=jnp.float32).astype(_DT)
    r2_ref[...] = jnp.dot(h, wr_ref[...], preferred_element_type=jnp.float32)
    inv_ref[...] = inv

  return pl.pallas_call(
      body,
      grid=(_N // _BR,),
      in_specs=[pl.BlockSpec((2, _BR, _W1), lambda i: (0, i, 0)),
                pl.BlockSpec((_BR, _DH), lambda i: (i, 0)),
                pl.BlockSpec((1, _DH), lambda i: (0, 0)),
                pl.BlockSpec((_DH, _DOUT), lambda i: (0, 0)),
                pl.BlockSpec((_DH, _DOUT), lambda i: (0, 0))],
      out_specs=(pl.BlockSpec((_BR, _W2), lambda i: (i, 0)),
                 pl.BlockSpec((_BR, _DOUT), lambda i: (i, 0)),
                 pl.BlockSpec((_BR, 1), lambda i: (i, 0))),
      out_shape=(jax.ShapeDtypeStruct((_N, _W2), _DT),
                 jax.ShapeDtypeStruct((_N, _DOUT), jnp.float32),
                 jax.ShapeDtypeStruct((_N, 1), jnp.float32)),
  )(aggp, r1, b1.reshape(1, _DH), Wl2, Wr2)


def _tc_post(aggp2, r2, inv, b2):
  def body(a_ref, r2_ref, inv_ref, b2_ref, out_ref):
    acc = (a_ref[0] + a_ref[1]).astype(jnp.float32)
    o = acc * inv_ref[...] + b2_ref[...] + r2_ref[...]
    nrm = jnp.sqrt(jnp.sum(o * o, axis=1, keepdims=True))
    o = o / jnp.maximum(nrm, 1e-12)
    out_ref[...] = jnp.where(o > 0.0, o, jnp.exp(jnp.minimum(o, 0.0)) - 1.0)

  return pl.pallas_call(
      body,
      grid=(_N // _BR,),
      in_specs=[pl.BlockSpec((2, _BR, _W2), lambda i: (0, i, 0)),
                pl.BlockSpec((_BR, _DOUT), lambda i: (i, 0)),
                pl.BlockSpec((_BR, 1), lambda i: (i, 0)),
                pl.BlockSpec((1, _DOUT), lambda i: (0, 0))],
      out_specs=pl.BlockSpec((_BR, _DOUT), lambda i: (i, 0)),
      out_shape=jax.ShapeDtypeStruct((_N, _DOUT), jnp.float32),
  )(aggp2, r2, inv, b2.reshape(1, _DOUT))


def kernel(x, edge_index, Wl1, b1, Wr1, Wl2, b2, Wr2):
  ei = edge_index.reshape(2, _NTILES, _NCH, _CB)
  table1, r1 = _tc_pre(x, Wl1, Wr1)
  aggp1 = _make_sc_segsum(_W1)(table1, ei)
  table2, r2, inv = _tc_mid(aggp1, r1, b1, Wl2, Wr2)
  aggp2 = _make_sc_segsum(_W2)(table2, ei)
  return _tc_post(aggp2, r2, inv, b2)


# speedup vs baseline: 1.0745x; 1.0156x over previous
import functools

import jax
from jax import lax
import jax.numpy as jnp
from jax.experimental import pallas as pl
from jax.experimental.pallas import tpu as pltpu
from jax.experimental.pallas import tpu_sc as plsc

_N = 10000
_E = 320000
_DIN = 128
_DH = 64
_DOUT = 64

_W1 = 96
_W2 = 64
_DT = jnp.bfloat16
_NTILES = 32
_EPT = _E // _NTILES
_CB = 125
_NCH = _EPT // _CB
_NBUF = 8
_NPAD = 10000
_RPT = _NPAD // 16
_ZR = 125
_BR = 2000


@functools.cache
def _make_sc_segsum(W):
  mesh = plsc.VectorSubcoreMesh(core_axis_name="c", subcore_axis_name="s")

  @functools.partial(
      pl.kernel,
      out_type=jax.ShapeDtypeStruct((2, _NPAD, W), _DT),
      mesh=mesh,
      compiler_params=pltpu.CompilerParams(use_tc_tiling_on_sc=False),
      scratch_types=[
          pltpu.VMEM((_NCH, _CB), jnp.int32),
          pltpu.VMEM((_NCH, _CB), jnp.int32),
          pltpu.VMEM((_CB, W), _DT),
          pltpu.VMEM((_CB, W), _DT),
          pltpu.VMEM((_CB, W), _DT),
          pltpu.VMEM((_CB, W), _DT),
          pltpu.VMEM((_CB, W), _DT),
          pltpu.VMEM((_CB, W), _DT),
          pltpu.VMEM((_CB, W), _DT),
          pltpu.VMEM((_CB, W), _DT),
          pltpu.VMEM((_ZR, W), _DT),
          pltpu.VMEM_SHARED((_NPAD, W), _DT),
          pltpu.SemaphoreType.DMA,
          pltpu.SemaphoreType.DMA,
          pltpu.SemaphoreType.DMA,
          pltpu.SemaphoreType.DMA,
          pltpu.SemaphoreType.DMA,
          pltpu.SemaphoreType.DMA,
          pltpu.SemaphoreType.DMA,
          pltpu.SemaphoreType.DMA,
          pltpu.SemaphoreType.DMA,
          pltpu.SemaphoreType.DMA,
          pltpu.SemaphoreType.DMA,
          pltpu.SemaphoreType.DMA,
          pltpu.SemaphoreType.DMA,
          pltpu.SemaphoreType.DMA,
          pltpu.SemaphoreType.DMA,
          pltpu.SemaphoreType.DMA,
      ],
  )
  def sc(table, ei, out, srcv, dstv, r0, r1, r2, r3, r4, r5, r6, r7, zb, acc,
         g0, g1, g2, g3, g4, g5, g6, g7, t0, t1, t2, t3, t4, t5, t6, t7):
    rows = (r0, r1, r2, r3, r4, r5, r6, r7)
    gsems = (g0, g1, g2, g3, g4, g5, g6, g7)
    ssems = (t0, t1, t2, t3, t4, t5, t6, t7)
    cid = lax.axis_index("c")
    sid = lax.axis_index("s")
    wid = sid * 2 + cid

    pltpu.sync_copy(ei.at[0, wid], srcv)
    pltpu.sync_copy(ei.at[1, wid], dstv)

    def gather(c, b):
      pltpu.async_copy(table.at[srcv.at[c]], rows[b], gsems[b])

    for b in range(_NBUF):
      gather(b, b)

    zvec = jnp.zeros((32,), _DT)

    def zrow(i, carry):
      for j in range(W // 32):
        zb[i, pl.ds(j * 32, 32)] = zvec
      return carry

    lax.fori_loop(0, _ZR, zrow, 0)
    row0 = sid * _RPT
    for k in range(_RPT // _ZR):
      pltpu.sync_copy(zb, acc.at[pl.ds(row0 + k * _ZR, _ZR)])
    plsc.subcore_barrier()

    def scatter(c, b):
      pltpu.make_async_copy(table.at[srcv.at[c]], rows[b], gsems[b]).wait()
      pltpu.async_copy(rows[b], acc.at[dstv.at[c]], ssems[b], add=True)

    def drain(c, b):
      pltpu.make_async_copy(rows[b], acc.at[dstv.at[c]], ssems[b]).wait()

    def block(j, carry):
      for b in range(_NBUF):
        scatter(j * _NBUF + b, b)
      for b in range(_NBUF):
        drain(j * _NBUF + b, b)
        gather((j + 1) * _NBUF + b, b)
      return carry

    nblk = _NCH // _NBUF
    lax.fori_loop(0, nblk - 1, block, 0)
    for b in range(_NBUF):
      scatter((nblk - 1) * _NBUF + b, b)
    for b in range(_NBUF):
      drain((nblk - 1) * _NBUF + b, b)
    plsc.subcore_barrier()

    for k in range(_RPT // _ZR):
      r = row0 + k * _ZR
      pltpu.sync_copy(acc.at[pl.ds(r, _ZR)], zb)
      pltpu.sync_copy(zb, out.at[cid, pl.ds(r, _ZR)])

  return sc


def _tc_pre(x, Wl1, Wr1):
  def body(x_ref, wl_ref, wr_ref, t_ref, r_ref):
    xv = x_ref[...]
    p = jnp.dot(xv, wl_ref[...], preferred_element_type=jnp.float32)
    t_ref[...] = jnp.concatenate(
        [p.astype(_DT), jnp.ones((_BR, _W1 - _DH), _DT)], axis=1)
    r_ref[...] = jnp.dot(xv, wr_ref[...], preferred_element_type=jnp.float32)

  return pl.pallas_call(
      body,
      grid=(_N // _BR,),
      in_specs=[pl.BlockSpec((_BR, _DIN), lambda i: (i, 0)),
                pl.BlockSpec((_DIN, _DH), lambda i: (0, 0)),
                pl.BlockSpec((_DIN, _DH), lambda i: (0, 0))],
      out_specs=(pl.BlockSpec((_BR, _W1), lambda i: (i, 0)),
                 pl.BlockSpec((_BR, _DH), lambda i: (i, 0))),
      out_shape=(jax.ShapeDtypeStruct((_N, _W1), _DT),
                 jax.ShapeDtypeStruct((_N, _DH), jnp.float32)),
  )(x, Wl1, Wr1)


def _tc_mid(aggp, r1, b1, Wl2, Wr2):
  def body(a_ref, r1_ref, b1_ref, wl_ref, wr_ref, t2_ref, r2_ref, inv_ref):
    acc = (a_ref[0] + a_ref[1]).astype(jnp.float32)
    cnt = acc[:, _DH:_DH + 1]
    inv = 1.0 / jnp.maximum(cnt, 1.0)
    o = acc[:, :_DH] * inv + b1_ref[...] + r1_ref[...]
    nrm = jnp.sqrt(jnp.sum(o * o, axis=1, keepdims=True))
    h = jnp.maximum(o / jnp.maximum(nrm, 1e-12), 0.0)
    t2_ref[...] = jnp.dot(
        h, wl_ref[...], preferred_element_type=jnp.float32).astype(_DT)
    r2_ref[...] = jnp.dot(h, wr_ref[...], preferred_element_type=jnp.float32)
    inv_ref[...] = inv

  return pl.pallas_call(
      body,
      grid=(_N // _BR,),
      in_specs=[pl.BlockSpec((2, _BR, _W1), lambda i: (0, i, 0)),
                pl.BlockSpec((_BR, _DH), lambda i: (i, 0)),
                pl.BlockSpec((1, _DH), lambda i: (0, 0)),
                pl.BlockSpec((_DH, _DOUT), lambda i: (0, 0)),
                pl.BlockSpec((_DH, _DOUT), lambda i: (0, 0))],
      out_specs=(pl.BlockSpec((_BR, _W2), lambda i: (i, 0)),
                 pl.BlockSpec((_BR, _DOUT), lambda i: (i, 0)),
                 pl.BlockSpec((_BR, 1), lambda i: (i, 0))),
      out_shape=(jax.ShapeDtypeStruct((_N, _W2), _DT),
                 jax.ShapeDtypeStruct((_N, _DOUT), jnp.float32),
                 jax.ShapeDtypeStruct((_N, 1), jnp.float32)),
  )(aggp, r1, b1.reshape(1, _DH), Wl2, Wr2)


def _tc_post(aggp2, r2, inv, b2):
  def body(a_ref, r2_ref, inv_ref, b2_ref, out_ref):
    acc = (a_ref[0] + a_ref[1]).astype(jnp.float32)
    o = acc * inv_ref[...] + b2_ref[...] + r2_ref[...]
    nrm = jnp.sqrt(jnp.sum(o * o, axis=1, keepdims=True))
    o = o / jnp.maximum(nrm, 1e-12)
    out_ref[...] = jnp.where(o > 0.0, o, jnp.exp(jnp.minimum(o, 0.0)) - 1.0)

  return pl.pallas_call(
      body,
      grid=(_N // _BR,),
      in_specs=[pl.BlockSpec((2, _BR, _W2), lambda i: (0, i, 0)),
                pl.BlockSpec((_BR, _DOUT), lambda i: (i, 0)),
                pl.BlockSpec((_BR, 1), lambda i: (i, 0)),
                pl.BlockSpec((1, _DOUT), lambda i: (0, 0))],
      out_specs=pl.BlockSpec((_BR, _DOUT), lambda i: (i, 0)),
      out_shape=jax.ShapeDtypeStruct((_N, _DOUT), jnp.float32),
  )(aggp2, r2, inv, b2.reshape(1, _DOUT))


def kernel(x, edge_index, Wl1, b1, Wr1, Wl2, b2, Wr2):
  ei = edge_index.reshape(2, _NTILES, _NCH, _CB)
  table1, r1 = _tc_pre(x, Wl1, Wr1)
  aggp1 = _make_sc_segsum(_W1)(table1, ei)
  table2, r2, inv = _tc_mid(aggp1, r1, b1, Wl2, Wr2)
  aggp2 = _make_sc_segsum(_W2)(table2, ei)
  return _tc_post(aggp2, r2, inv, b2)


# speedup vs baseline: 1.0848x; 1.0095x over previous
import functools

import jax
from jax import lax
import jax.numpy as jnp
from jax.experimental import pallas as pl
from jax.experimental.pallas import tpu as pltpu
from jax.experimental.pallas import tpu_sc as plsc

_N = 10000
_E = 320000
_DIN = 128
_DH = 64
_DOUT = 64

_W1 = 96
_W2 = 64
_DT = jnp.bfloat16
_NTILES = 32
_EPT = _E // _NTILES
_CB = 125
_NCH = _EPT // _CB
_NBUF = 8
_NPAD = 10000
_RPT = _NPAD // 16
_ZR = 125
_BR = 2000


@functools.cache
def _make_sc_segsum(W):
  mesh = plsc.VectorSubcoreMesh(core_axis_name="c", subcore_axis_name="s")

  @functools.partial(
      pl.kernel,
      out_type=jax.ShapeDtypeStruct((2, _NPAD, W), _DT),
      mesh=mesh,
      compiler_params=pltpu.CompilerParams(use_tc_tiling_on_sc=False),
      scratch_types=[
          pltpu.VMEM((_NCH, _CB), jnp.int32),
          pltpu.VMEM((_NCH, _CB), jnp.int32),
          pltpu.VMEM((_CB, W), _DT),
          pltpu.VMEM((_CB, W), _DT),
          pltpu.VMEM((_CB, W), _DT),
          pltpu.VMEM((_CB, W), _DT),
          pltpu.VMEM((_CB, W), _DT),
          pltpu.VMEM((_CB, W), _DT),
          pltpu.VMEM((_CB, W), _DT),
          pltpu.VMEM((_CB, W), _DT),
          pltpu.VMEM((_ZR, W), _DT),
          pltpu.VMEM((_ZR, W), _DT),
          pltpu.VMEM_SHARED((_NPAD, W), _DT),
          pltpu.SemaphoreType.DMA,
          pltpu.SemaphoreType.DMA,
          pltpu.SemaphoreType.DMA,
          pltpu.SemaphoreType.DMA,
          pltpu.SemaphoreType.DMA,
          pltpu.SemaphoreType.DMA,
          pltpu.SemaphoreType.DMA,
          pltpu.SemaphoreType.DMA,
          pltpu.SemaphoreType.DMA,
          pltpu.SemaphoreType.DMA,
          pltpu.SemaphoreType.DMA,
          pltpu.SemaphoreType.DMA,
          pltpu.SemaphoreType.DMA,
          pltpu.SemaphoreType.DMA,
          pltpu.SemaphoreType.DMA,
          pltpu.SemaphoreType.DMA,
      ],
  )
  def sc(table, ei, out, srcv, dstv, r0, r1, r2, r3, r4, r5, r6, r7, zb, zb2,
         acc, g0, g1, g2, g3, g4, g5, g6, g7, t0, t1, t2, t3, t4, t5, t6, t7):
    rows = (r0, r1, r2, r3, r4, r5, r6, r7)
    gsems = (g0, g1, g2, g3, g4, g5, g6, g7)
    ssems = (t0, t1, t2, t3, t4, t5, t6, t7)
    cid = lax.axis_index("c")
    sid = lax.axis_index("s")
    wid = sid * 2 + cid

    pltpu.sync_copy(ei.at[0, wid], srcv)
    pltpu.sync_copy(ei.at[1, wid], dstv)

    def gather(c, b):
      pltpu.async_copy(table.at[srcv.at[c]], rows[b], gsems[b])

    for b in range(_NBUF):
      gather(b, b)

    zvec = jnp.zeros((32,), _DT)

    def zrow(i, carry):
      for j in range(W // 32):
        zb[i, pl.ds(j * 32, 32)] = zvec
      return carry

    lax.fori_loop(0, _ZR, zrow, 0)
    row0 = sid * _RPT
    for k in range(_RPT // _ZR):
      pltpu.sync_copy(zb, acc.at[pl.ds(row0 + k * _ZR, _ZR)])
    plsc.subcore_barrier()

    def scatter(c, b):
      pltpu.make_async_copy(table.at[srcv.at[c]], rows[b], gsems[b]).wait()
      pltpu.async_copy(rows[b], acc.at[dstv.at[c]], ssems[b], add=True)

    def drain(c, b):
      pltpu.make_async_copy(rows[b], acc.at[dstv.at[c]], ssems[b]).wait()

    def block(j, carry):
      for b in range(_NBUF):
        scatter(j * _NBUF + b, b)
      for b in range(_NBUF):
        drain(j * _NBUF + b, b)
        gather((j + 1) * _NBUF + b, b)
      return carry

    nblk = _NCH // _NBUF
    lax.fori_loop(0, nblk - 1, block, 0)
    for b in range(_NBUF):
      scatter((nblk - 1) * _NBUF + b, b)
    for b in range(_NBUF):
      drain((nblk - 1) * _NBUF + b, b)
    plsc.subcore_barrier()

    bufs = (zb, zb2)
    nko = _RPT // _ZR
    for k in range(nko):
      r = row0 + k * _ZR
      buf, sem = bufs[k % 2], gsems[k % 2]
      if k >= 2:
        pltpu.make_async_copy(
            buf, out.at[cid, pl.ds(r - 2 * _ZR, _ZR)], sem).wait()
      pltpu.sync_copy(acc.at[pl.ds(r, _ZR)], buf)
      pltpu.async_copy(buf, out.at[cid, pl.ds(r, _ZR)], sem)
    for k in range(max(nko - 2, 0), nko):
      r = row0 + k * _ZR
      pltpu.make_async_copy(
          bufs[k % 2], out.at[cid, pl.ds(r, _ZR)], gsems[k % 2]).wait()

  return sc


def _tc_pre(x, Wl1, Wr1):
  def body(x_ref, wl_ref, wr_ref, t_ref, r_ref):
    xv = x_ref[...]
    p = jnp.dot(xv, wl_ref[...], preferred_element_type=jnp.float32)
    t_ref[...] = jnp.concatenate(
        [p.astype(_DT), jnp.ones((_BR, _W1 - _DH), _DT)], axis=1)
    r_ref[...] = jnp.dot(xv, wr_ref[...], preferred_element_type=jnp.float32)

  return pl.pallas_call(
      body,
      grid=(_N // _BR,),
      in_specs=[pl.BlockSpec((_BR, _DIN), lambda i: (i, 0)),
                pl.BlockSpec((_DIN, _DH), lambda i: (0, 0)),
                pl.BlockSpec((_DIN, _DH), lambda i: (0, 0))],
      out_specs=(pl.BlockSpec((_BR, _W1), lambda i: (i, 0)),
                 pl.BlockSpec((_BR, _DH), lambda i: (i, 0))),
      out_shape=(jax.ShapeDtypeStruct((_N, _W1), _DT),
                 jax.ShapeDtypeStruct((_N, _DH), jnp.float32)),
  )(x, Wl1, Wr1)


def _tc_mid(aggp, r1, b1, Wl2, Wr2):
  def body(a_ref, r1_ref, b1_ref, wl_ref, wr_ref, t2_ref, r2_ref, inv_ref):
    acc = (a_ref[0] + a_ref[1]).astype(jnp.float32)
    cnt = acc[:, _DH:_DH + 1]
    inv = 1.0 / jnp.maximum(cnt, 1.0)
    o = acc[:, :_DH] * inv + b1_ref[...] + r1_ref[...]
    nrm = jnp.sqrt(jnp.sum(o * o, axis=1, keepdims=True))
    h = jnp.maximum(o / jnp.maximum(nrm, 1e-12), 0.0)
    t2_ref[...] = jnp.dot(
        h, wl_ref[...], preferred_element_type=jnp.float32).astype(_DT)
    r2_ref[...] = jnp.dot(h, wr_ref[...], preferred_element_type=jnp.float32)
    inv_ref[...] = inv

  return pl.pallas_call(
      body,
      grid=(_N // _BR,),
      in_specs=[pl.BlockSpec((2, _BR, _W1), lambda i: (0, i, 0)),
                pl.BlockSpec((_BR, _DH), lambda i: (i, 0)),
                pl.BlockSpec((1, _DH), lambda i: (0, 0)),
                pl.BlockSpec((_DH, _DOUT), lambda i: (0, 0)),
                pl.BlockSpec((_DH, _DOUT), lambda i: (0, 0))],
      out_specs=(pl.BlockSpec((_BR, _W2), lambda i: (i, 0)),
                 pl.BlockSpec((_BR, _DOUT), lambda i: (i, 0)),
                 pl.BlockSpec((_BR, 1), lambda i: (i, 0))),
      out_shape=(jax.ShapeDtypeStruct((_N, _W2), _DT),
                 jax.ShapeDtypeStruct((_N, _DOUT), jnp.float32),
                 jax.ShapeDtypeStruct((_N, 1), jnp.float32)),
  )(aggp, r1, b1.reshape(1, _DH), Wl2, Wr2)


def _tc_post(aggp2, r2, inv, b2):
  def body(a_ref, r2_ref, inv_ref, b2_ref, out_ref):
    acc = (a_ref[0] + a_ref[1]).astype(jnp.float32)
    o = acc * inv_ref[...] + b2_ref[...] + r2_ref[...]
    nrm = jnp.sqrt(jnp.sum(o * o, axis=1, keepdims=True))
    o = o / jnp.maximum(nrm, 1e-12)
    out_ref[...] = jnp.where(o > 0.0, o, jnp.exp(jnp.minimum(o, 0.0)) - 1.0)

  return pl.pallas_call(
      body,
      grid=(_N // _BR,),
      in_specs=[pl.BlockSpec((2, _BR, _W2), lambda i: (0, i, 0)),
                pl.BlockSpec((_BR, _DOUT), lambda i: (i, 0)),
                pl.BlockSpec((_BR, 1), lambda i: (i, 0)),
                pl.BlockSpec((1, _DOUT), lambda i: (0, 0))],
      out_specs=pl.BlockSpec((_BR, _DOUT), lambda i: (i, 0)),
      out_shape=jax.ShapeDtypeStruct((_N, _DOUT), jnp.float32),
  )(aggp2, r2, inv, b2.reshape(1, _DOUT))


def kernel(x, edge_index, Wl1, b1, Wr1, Wl2, b2, Wr2):
  ei = edge_index.reshape(2, _NTILES, _NCH, _CB)
  table1, r1 = _tc_pre(x, Wl1, Wr1)
  aggp1 = _make_sc_segsum(_W1)(table1, ei)
  table2, r2, inv = _tc_mid(aggp1, r1, b1, Wl2, Wr2)
  aggp2 = _make_sc_segsum(_W2)(table2, ei)
  return _tc_post(aggp2, r2, inv, b2)
